# baseline (speedup 1.0000x reference)
import functools

import jax
import jax.numpy as jnp
from jax import lax
from jax.experimental import pallas as pl
from jax.experimental.pallas import tpu as pltpu
from jax.experimental.pallas import tpu_sc as plsc

VOCAB = 1000000
K = 16
B = 4096
F = 26

NC = 2
NS = 16
NW = NC * NS
RPW = B // NW
RPC = 4
CS = RPC * F
CSP = 112
CH = RPW // RPC
WV = VOCAB // K


def _full(x):
  return jnp.full((K,), x, dtype=jnp.int32)


def _fm_body(idx_hbm, idxhi_hbm, vals_hbm, bias_hbm, w_hbm, emb_hbm, out_hbm,
             idx_v, idxhi_v, vals_v, bias_v, rows_v, wbuf_v, lw_v, out_v, sem):
  c = lax.axis_index("c")
  s = lax.axis_index("s")
  wid = s * NC + c

  pltpu.sync_copy(idx_hbm.at[wid], idx_v)
  pltpu.sync_copy(idxhi_hbm.at[wid], idxhi_v)
  pltpu.sync_copy(vals_hbm.at[wid], vals_v)
  pltpu.sync_copy(bias_hbm, bias_v)
  bias16 = bias_v[...] * (1.0 / 16.0)

  lane = lax.iota(jnp.int32, 16)
  mask0 = lane == 0
  tail10 = (lane < 10).astype(jnp.float32)

  def chunk_body(j, _):
    cp_e = pltpu.async_copy(emb_hbm.at[idx_v.at[j]], rows_v, sem)
    cp_w = pltpu.async_copy(w_hbm.at[idxhi_v.at[j]], wbuf_v, sem)
    cp_e.wait()
    cp_w.wait()

    jvec = jnp.full((K,), j, dtype=jnp.int32)
    for g in range(CSP // 16):
      cols = lane + g * 16
      if g == CSP // 16 - 1:
        m = cols < CS
        sel = jnp.float32(0)
      else:
        m = None
        sel = None
      iv = plsc.load_gather(idx_v, [jvec, cols], mask=m)
      vv = plsc.load_gather(vals_v, [jvec, cols], mask=m)
      ws = plsc.load_gather(wbuf_v, [cols, iv & 15], mask=m)
      lw = vv * ws
      if m is not None:
        lw = jnp.where(m, lw, sel)
      lw_v[pl.ds(g * 16, 16)] = lw

    for r in range(RPC):
      xv = jnp.zeros((K,), jnp.float32)
      ss = jnp.zeros((K,), jnp.float32)
      for f in range(F):
        i = r * F + f
        e = rows_v[i]
        v = plsc.load_gather(vals_v, [jvec, _full(i)])
        t = v * e
        xv = xv + t
        ss = ss + t * t
      s1 = lw_v[pl.ds(r * F, 16)]
      s2 = lw_v[pl.ds(r * F + 16, 16)] * tail10
      res = 0.5 * (xv * xv - ss) + s1 + s2 + bias16
      total = jnp.sum(res)
      row = j * RPC + r
      plsc.store_scatter(out_v, [_full(row)],
                         jnp.full((K,), total, jnp.float32), mask=mask0)
    return 0

  lax.fori_loop(0, CH, chunk_body, 0)
  pltpu.sync_copy(out_v, out_hbm.at[wid])


@functools.partial(jax.jit, static_argnames=())
def kernel(indices, values, bias, weight, embedding):
  idx = indices.astype(jnp.int32)
  idx3 = idx.reshape(NW, CH, CS)
  idxhi3 = (idx >> 4).reshape(NW, CH, CS)
  vals3 = values.astype(jnp.float32).reshape(NW, CH, CS)
  bias16 = jnp.broadcast_to(bias.astype(jnp.float32), (16,))
  w16 = weight.reshape(WV, K)

  mesh = plsc.VectorSubcoreMesh(
      core_axis_name="c", subcore_axis_name="s", num_cores=NC, num_subcores=NS)
  fm = pl.kernel(
      _fm_body,
      out_type=jax.ShapeDtypeStruct((NW, RPW), jnp.float32),
      mesh=mesh,
      compiler_params=pltpu.CompilerParams(
          needs_layout_passes=False, use_tc_tiling_on_sc=False),
      scratch_types=[
          pltpu.VMEM((CH, CS), jnp.int32),
          pltpu.VMEM((CH, CS), jnp.int32),
          pltpu.VMEM((CH, CS), jnp.float32),
          pltpu.VMEM((16,), jnp.float32),
          pltpu.VMEM((CS, K), jnp.float32),
          pltpu.VMEM((CS, K), jnp.float32),
          pltpu.VMEM((CSP,), jnp.float32),
          pltpu.VMEM((RPW,), jnp.float32),
          pltpu.SemaphoreType.DMA,
      ],
  )
  out = fm(idx3, idxhi3, vals3, bias16, w16, embedding)
  return out.reshape(B, 1)


# speedup vs baseline: 1.0458x; 1.0458x over previous
import functools

import jax
import jax.numpy as jnp
from jax import lax
from jax.experimental import pallas as pl
from jax.experimental.pallas import tpu as pltpu
from jax.experimental.pallas import tpu_sc as plsc

VOCAB = 1000000
K = 16
B = 4096
F = 26

NC = 2
NS = 16
NW = NC * NS
RPW = B // NW
RPC = 4
CS = RPC * F
CSP = 112
CH = RPW // RPC
WV = VOCAB // K


def _full(x):
  return jnp.full((K,), x, dtype=jnp.int32)


NBUF = 4


def _fm_body(idx_hbm, idxhi_hbm, vals_hbm, bias_hbm, w_hbm, emb_hbm, out_hbm,
             idx_v, idxhi_v, vals_v, bias_v, rows_v, wbuf_v, lw_v, out_v,
             *sems):
  c = lax.axis_index("c")
  s = lax.axis_index("s")
  wid = s * NC + c

  pltpu.sync_copy(idx_hbm.at[wid], idx_v)
  pltpu.sync_copy(idxhi_hbm.at[wid], idxhi_v)
  pltpu.sync_copy(vals_hbm.at[wid], vals_v)
  pltpu.sync_copy(bias_hbm, bias_v)
  bias16 = bias_v[...] * (1.0 / 16.0)

  lane = lax.iota(jnp.int32, 16)
  mask0 = lane == 0
  tail10 = (lane < 10).astype(jnp.float32)

  def fire(j, b):
    pltpu.async_copy(emb_hbm.at[idx_v.at[j]], rows_v.at[b], sems[b])
    pltpu.async_copy(w_hbm.at[idxhi_v.at[j]], wbuf_v.at[b], sems[b])

  def wait(j, b):
    pltpu.make_async_copy(emb_hbm.at[idx_v.at[j]], rows_v.at[b],
                          sems[b]).wait()
    pltpu.make_async_copy(w_hbm.at[idxhi_v.at[j]], wbuf_v.at[b],
                          sems[b]).wait()

  def compute(j, b):
    jvec = jnp.full((K,), j, dtype=jnp.int32)
    for g in range(CSP // 16):
      cols = lane + g * 16
      if g == CSP // 16 - 1:
        m = cols < CS
      else:
        m = None
      iv = plsc.load_gather(idx_v, [jvec, cols], mask=m)
      vv = plsc.load_gather(vals_v, [jvec, cols], mask=m)
      ws = plsc.load_gather(wbuf_v.at[b], [cols, iv & 15], mask=m)
      lw = vv * ws
      if m is not None:
        lw = jnp.where(m, lw, jnp.float32(0))
      lw_v[pl.ds(g * 16, 16)] = lw

    for r in range(RPC):
      xv = jnp.zeros((K,), jnp.float32)
      ss = jnp.zeros((K,), jnp.float32)
      for f in range(F):
        i = r * F + f
        e = rows_v[b, i]
        v = plsc.load_gather(vals_v, [jvec, _full(i)])
        t = v * e
        xv = xv + t
        ss = ss + t * t
      s1 = lw_v[pl.ds(r * F, 16)]
      s2 = lw_v[pl.ds(r * F + 16, 16)] * tail10
      res = 0.5 * (xv * xv - ss) + s1 + s2 + bias16
      total = jnp.sum(res)
      row = j * RPC + r
      plsc.store_scatter(out_v, [_full(row)],
                         jnp.full((K,), total, jnp.float32), mask=mask0)

  for p in range(NBUF - 1):
    fire(jnp.int32(p), p)

  def group_body(g, _):
    for bb in range(NBUF):
      j = g * NBUF + bb
      jn = j + NBUF - 1

      @pl.when(jn < CH)
      def _():
        fire(jn, (bb + NBUF - 1) % NBUF)

      wait(j, bb)
      compute(j, bb)
    return 0

  lax.fori_loop(0, CH // NBUF, group_body, 0)
  pltpu.sync_copy(out_v, out_hbm.at[wid])


@functools.partial(jax.jit, static_argnames=())
def kernel(indices, values, bias, weight, embedding):
  idx = indices.astype(jnp.int32)
  idx3 = idx.reshape(NW, CH, CS)
  idxhi3 = (idx >> 4).reshape(NW, CH, CS)
  vals3 = values.astype(jnp.float32).reshape(NW, CH, CS)
  bias16 = jnp.broadcast_to(bias.astype(jnp.float32), (16,))
  w16 = weight.reshape(WV, K)

  mesh = plsc.VectorSubcoreMesh(
      core_axis_name="c", subcore_axis_name="s", num_cores=NC, num_subcores=NS)
  fm = pl.kernel(
      _fm_body,
      out_type=jax.ShapeDtypeStruct((NW, RPW), jnp.float32),
      mesh=mesh,
      compiler_params=pltpu.CompilerParams(
          needs_layout_passes=False, use_tc_tiling_on_sc=False),
      scratch_types=[
          pltpu.VMEM((CH, CS), jnp.int32),
          pltpu.VMEM((CH, CS), jnp.int32),
          pltpu.VMEM((CH, CS), jnp.float32),
          pltpu.VMEM((16,), jnp.float32),
          pltpu.VMEM((NBUF, CS, K), jnp.float32),
          pltpu.VMEM((NBUF, CS, K), jnp.float32),
          pltpu.VMEM((CSP,), jnp.float32),
          pltpu.VMEM((RPW,), jnp.float32),
      ] + [pltpu.SemaphoreType.DMA] * NBUF,
  )
  out = fm(idx3, idxhi3, vals3, bias16, w16, embedding)
  return out.reshape(B, 1)
